# Initial kernel scaffold; baseline (speedup 1.0000x reference)
import functools

import jax
import jax.numpy as jnp
from jax import lax
from jax.experimental import pallas as pl
from jax.experimental.pallas import tpu as pltpu
from jax.experimental.pallas import tpu_sc as plsc

NUM_CORES = 2
NUM_SUBCORES = 16
NW = NUM_CORES * NUM_SUBCORES


@functools.lru_cache(maxsize=None)
def _make_gather(N, V, D, C):
    per_w = N // NW
    nchunk = per_w // C
    mesh = plsc.VectorSubcoreMesh(core_axis_name="c", subcore_axis_name="s")

    @functools.partial(
        pl.kernel,
        out_type=jax.ShapeDtypeStruct((N, D), jnp.float32),
        mesh=mesh,
        scratch_types=[
            pltpu.VMEM((C,), jnp.int32),
            pltpu.VMEM((C, D), jnp.float32),
            pltpu.SemaphoreType.DMA,
        ],
    )
    def k(table_hbm, idx_hbm, out_hbm, idx_v, rows_v, sem):
        wid = lax.axis_index("s") * NUM_CORES + lax.axis_index("c")
        w_base = wid * per_w

        def body(i, carry):
            base = w_base + i * C
            pltpu.sync_copy(idx_hbm.at[pl.ds(base, C)], idx_v)
            pltpu.async_copy(table_hbm.at[idx_v], rows_v, sem).wait()
            pltpu.sync_copy(rows_v, out_hbm.at[pl.ds(base, C)])
            return carry

        lax.fori_loop(0, nchunk, body, 0)

    return k


def kernel(category_inputs, tables):
    B, F = category_inputs.shape
    _, V, D = tables.shape
    flat_idx = (category_inputs.astype(jnp.int32)
                + jnp.arange(F, dtype=jnp.int32)[None, :] * V).reshape(-1)
    flat_tab = tables.reshape(F * V, D)
    out = _make_gather(B * F, V, D, 1664)(flat_tab, flat_idx)
    return out.reshape(B, F * D)


# speedup vs baseline: 1.2092x; 1.2092x over previous
import functools

import jax
import jax.numpy as jnp
from jax import lax
from jax.experimental import pallas as pl
from jax.experimental.pallas import tpu as pltpu
from jax.experimental.pallas import tpu_sc as plsc

NUM_CORES = 2
NUM_SUBCORES = 16
NW = NUM_CORES * NUM_SUBCORES


@functools.lru_cache(maxsize=None)
def _make_gather(N, V, D, C):
    per_w = N // NW
    nchunk = per_w // C
    mesh = plsc.VectorSubcoreMesh(core_axis_name="c", subcore_axis_name="s")

    @functools.partial(
        pl.kernel,
        out_type=jax.ShapeDtypeStruct((N, D), jnp.float32),
        mesh=mesh,
        scratch_types=[
            pltpu.VMEM((C,), jnp.int32),
            pltpu.VMEM((C, D), jnp.float32),
            pltpu.SemaphoreType.DMA,
        ],
        compiler_params=pltpu.CompilerParams(use_tc_tiling_on_sc=False),
    )
    def k(table_hbm, idx_hbm, out_hbm, idx_v, rows_v, sem):
        wid = lax.axis_index("s") * NUM_CORES + lax.axis_index("c")
        w_base = wid * per_w

        def body(i, carry):
            base = w_base + i * C
            pltpu.sync_copy(idx_hbm.at[pl.ds(base, C)], idx_v)
            pltpu.async_copy(table_hbm.at[idx_v], rows_v, sem).wait()
            pltpu.sync_copy(rows_v, out_hbm.at[pl.ds(base, C)])
            return carry

        lax.fori_loop(0, nchunk, body, 0)

    return k


def kernel(category_inputs, tables):
    B, F = category_inputs.shape
    _, V, D = tables.shape
    flat_idx = (category_inputs.astype(jnp.int32)
                + jnp.arange(F, dtype=jnp.int32)[None, :] * V).reshape(-1)
    flat_tab = tables.reshape(F * V, D)
    out = _make_gather(B * F, V, D, 1664)(flat_tab, flat_idx)
    return out.reshape(B, F * D)


# speedup vs baseline: 3.1962x; 2.6432x over previous
import functools

import jax
import jax.numpy as jnp
from jax import lax
from jax.experimental import pallas as pl
from jax.experimental.pallas import tpu as pltpu
from jax.experimental.pallas import tpu_sc as plsc

NUM_CORES = 2
NUM_SUBCORES = 16
NW = NUM_CORES * NUM_SUBCORES
LANES = 16
HBI = 2048
UNROLL = 8


@functools.lru_cache(maxsize=None)
def _make_gather_t(FD, V, B, D):
    rows_per_w = FD // NW
    VH = (V // 2) // 128 * 128
    VR = V - VH
    n_chunks = B // HBI
    mesh = plsc.VectorSubcoreMesh(core_axis_name="c", subcore_axis_name="s")

    @functools.partial(
        pl.kernel,
        out_type=jax.ShapeDtypeStruct((FD, B), jnp.float32),
        mesh=mesh,
        scratch_types=[
            pltpu.VMEM((VH,), jnp.float32),
            pltpu.VMEM((VR,), jnp.float32),
            pltpu.VMEM((HBI,), jnp.int32),
            pltpu.VMEM((HBI,), jnp.int32),
            pltpu.VMEM((B,), jnp.float32),
            pltpu.SemaphoreType.DMA,
            pltpu.SemaphoreType.DMA,
            pltpu.SemaphoreType.DMA,
            pltpu.SemaphoreType.DMA,
            pltpu.SemaphoreType.DMA,
        ],
        compiler_params=pltpu.CompilerParams(needs_layout_passes=False),
    )
    def k(tab_hbm, idx_hbm, out_hbm, tab_lo, tab_hi, idx_a, idx_b, out_v,
          s_lo, s_hi, s_ia, s_ib, s_out):
        wid = lax.axis_index("s") * NUM_CORES + lax.axis_index("c")
        c0 = wid * rows_per_w
        iota = lax.iota(jnp.int32, LANES)
        idx_bufs = (idx_a, idx_b)
        idx_sems = (s_ia, s_ib)

        def start_idx(f, h, slot):
            pltpu.async_copy(idx_hbm.at[f, pl.ds(h * HBI, HBI)],
                             idx_bufs[slot], idx_sems[slot])

        def wait_idx(f, slot):
            pltpu.make_async_copy(idx_hbm.at[f, pl.ds(0, HBI)],
                                  idx_bufs[slot], idx_sems[slot]).wait()

        def sweep(f, first):
            for h in range(n_chunks):
                if h + 1 < n_chunks:
                    start_idx(f, h + 1, (h + 1) % 2)
                wait_idx(f, h % 2)
                buf = idx_bufs[h % 2]

                def body(kk, carry, h=h, buf=buf):
                    base = kk * (UNROLL * LANES)
                    for u in range(UNROLL):
                        off = base + u * LANES
                        iv = buf[pl.ds(off, LANES)]
                        if first:
                            m = iv < VH
                            g = plsc.load_gather(
                                tab_lo, [jnp.minimum(iv, VH - 1)], mask=m)
                            out_v[pl.ds(h * HBI + off, LANES)] = g
                        else:
                            m = iv >= VH
                            g = plsc.load_gather(
                                tab_hi, [jnp.maximum(iv - VH, 0)], mask=m)
                            plsc.store_scatter(
                                out_v, [h * HBI + off + iota], g, mask=m)
                    return carry

                lax.fori_loop(0, HBI // (UNROLL * LANES), body, 0)

        def row_body(j, carry):
            c = c0 + j
            f = c // D
            pltpu.async_copy(tab_hbm.at[c, pl.ds(VH, VR)], tab_hi, s_hi)
            start_idx(f, 0, 0)

            @pl.when(j > 0)
            def _():
                pltpu.make_async_copy(out_v, out_hbm.at[c], s_out).wait()

            pltpu.make_async_copy(tab_hbm.at[c, pl.ds(0, VH)],
                                  tab_lo, s_lo).wait()
            sweep(f, True)

            @pl.when(j < rows_per_w - 1)
            def _():
                pltpu.async_copy(tab_hbm.at[c + 1, pl.ds(0, VH)], tab_lo, s_lo)

            start_idx(f, 0, 0)
            pltpu.make_async_copy(tab_hbm.at[c, pl.ds(VH, VR)],
                                  tab_hi, s_hi).wait()
            sweep(f, False)

            pltpu.async_copy(out_v, out_hbm.at[c], s_out)
            return carry

        pltpu.async_copy(tab_hbm.at[c0, pl.ds(0, VH)], tab_lo, s_lo)
        lax.fori_loop(0, rows_per_w, row_body, 0)
        pltpu.make_async_copy(out_v, out_hbm.at[c0], s_out).wait()

    return k


def kernel(category_inputs, tables):
    B, F = category_inputs.shape
    _, V, D = tables.shape
    idx_t = category_inputs.astype(jnp.int32).T
    tab_t = jnp.transpose(tables, (0, 2, 1)).reshape(F * D, V)
    out_t = _make_gather_t(F * D, V, B, D)(tab_t, idx_t)
    return out_t.T


# speedup vs baseline: 6.1829x; 1.9345x over previous
import functools

import jax
import jax.numpy as jnp
from jax import lax
from jax.experimental import pallas as pl
from jax.experimental.pallas import tpu as pltpu
from jax.experimental.pallas import tpu_sc as plsc

NUM_CORES = 2
NUM_SUBCORES = 16
NW = NUM_CORES * NUM_SUBCORES
LANES = 16
UNROLL = 16


@functools.lru_cache(maxsize=None)
def _make_gather_t(FD, V, B, D):
    rows_per_w = FD // NW
    HB = 4096
    nh = B // HB
    mesh = plsc.VectorSubcoreMesh(core_axis_name="c", subcore_axis_name="s")

    @functools.partial(
        pl.kernel,
        out_type=jax.ShapeDtypeStruct((FD, B), jnp.float32),
        mesh=mesh,
        scratch_types=[
            pltpu.VMEM((V,), jnp.float32),
            pltpu.VMEM((HB,), jnp.int32),
            pltpu.VMEM((HB,), jnp.int32),
            pltpu.VMEM((HB,), jnp.float32),
            pltpu.VMEM((HB,), jnp.float32),
            pltpu.SemaphoreType.DMA,
            pltpu.SemaphoreType.DMA,
            pltpu.SemaphoreType.DMA,
            pltpu.SemaphoreType.DMA,
            pltpu.SemaphoreType.DMA,
        ],
        compiler_params=pltpu.CompilerParams(needs_layout_passes=False),
    )
    def k(tab_hbm, idx_hbm, out_hbm, tab_v, idx_a, idx_b, out_a, out_b,
          s_lo, s_hi, s_ia, s_ib, s_out):
        wid = lax.axis_index("s") * NUM_CORES + lax.axis_index("c")
        c0 = wid * rows_per_w
        idx_bufs = (idx_a, idx_b)
        idx_sems = (s_ia, s_ib)
        out_bufs = (out_a, out_b)

        def start_tab(c):
            pltpu.async_copy(tab_hbm.at[c], tab_v, s_lo)

        def wait_tab(c):
            pltpu.make_async_copy(tab_hbm.at[c], tab_v, s_lo).wait()

        def start_idx(f, h, slot):
            pltpu.async_copy(idx_hbm.at[f, pl.ds(h * HB, HB)],
                             idx_bufs[slot], idx_sems[slot])

        def wait_idx(slot):
            pltpu.make_async_copy(idx_hbm.at[0, pl.ds(0, HB)],
                                  idx_bufs[slot], idx_sems[slot]).wait()

        def wait_out(slot):
            pltpu.make_async_copy(out_bufs[slot],
                                  out_hbm.at[c0, pl.ds(0, HB)], s_out).wait()

        def row_body(j, carry):
            c = c0 + j
            f = c // D
            f_nxt = (c + 1) // D
            wait_tab(c)
            for h in range(nh):
                slot = h % 2
                if h + 1 < nh:
                    start_idx(f, h + 1, (h + 1) % 2)
                else:
                    @pl.when(j < rows_per_w - 1)
                    def _():
                        start_idx(f_nxt, 0, (h + 1) % 2)
                wait_idx(slot)
                if h < 2:
                    @pl.when(j > 0)
                    def _():
                        wait_out(slot)
                else:
                    wait_out(slot)
                buf = idx_bufs[slot]
                obuf = out_bufs[slot]

                def body(kk, carry3, buf=buf, obuf=obuf):
                    base = kk * (UNROLL * LANES)
                    for u in range(UNROLL):
                        off = base + u * LANES
                        iv = buf[pl.ds(off, LANES)]
                        obuf[pl.ds(off, LANES)] = plsc.load_gather(tab_v, [iv])
                    return carry3

                lax.fori_loop(0, HB // (UNROLL * LANES), body, 0)
                if h == nh - 1:
                    @pl.when(j < rows_per_w - 1)
                    def _():
                        start_tab(c + 1)
                pltpu.async_copy(obuf, out_hbm.at[c, pl.ds(h * HB, HB)], s_out)
            return carry

        start_tab(c0)
        start_idx(c0 // D, 0, 0)
        lax.fori_loop(0, rows_per_w, row_body, 0)
        wait_out(0)
        wait_out(1)

    return k


def kernel(category_inputs, tables):
    B, F = category_inputs.shape
    _, V, D = tables.shape
    idx_t = category_inputs.astype(jnp.int32).T
    tab_t = jnp.transpose(tables, (0, 2, 1)).reshape(F * D, V)
    out_t = _make_gather_t(F * D, V, B, D)(tab_t, idx_t)
    return out_t.T


# speedup vs baseline: 7.6423x; 1.2360x over previous
import functools

import jax
import jax.numpy as jnp
from jax import lax
from jax.experimental import pallas as pl
from jax.experimental.pallas import tpu as pltpu
from jax.experimental.pallas import tpu_sc as plsc

NUM_CORES = 2
NUM_SUBCORES = 16
NW = NUM_CORES * NUM_SUBCORES
LANES = 16
UNROLL = 16


@functools.lru_cache(maxsize=None)
def _make_gather_t(FD, V, B, D):
    rows_per_w = FD // NW
    HB = 4096
    nh = B // HB
    mesh = plsc.VectorSubcoreMesh(core_axis_name="c", subcore_axis_name="s")

    @functools.partial(
        pl.kernel,
        out_type=jax.ShapeDtypeStruct((FD, B), jnp.float32),
        mesh=mesh,
        scratch_types=[
            pltpu.VMEM((V,), jnp.float32),
            pltpu.VMEM((HB,), jnp.int32),
            pltpu.VMEM((HB,), jnp.int32),
            pltpu.VMEM((HB,), jnp.float32),
            pltpu.VMEM((HB,), jnp.float32),
            pltpu.SemaphoreType.DMA,
            pltpu.SemaphoreType.DMA,
            pltpu.SemaphoreType.DMA,
            pltpu.SemaphoreType.DMA,
            pltpu.SemaphoreType.DMA,
        ],
        compiler_params=pltpu.CompilerParams(needs_layout_passes=False),
    )
    def k(tab_hbm, idx_hbm, out_hbm, tab_v, idx_a, idx_b, out_a, out_b,
          s_lo, s_hi, s_ia, s_ib, s_out):
        wid = lax.axis_index("s") * NUM_CORES + lax.axis_index("c")
        c0 = wid * rows_per_w
        idx_bufs = (idx_a, idx_b)
        idx_sems = (s_ia, s_ib)
        out_bufs = (out_a, out_b)

        def start_tab(c):
            pltpu.async_copy(tab_hbm.at[c], tab_v, s_lo)

        def wait_tab(c):
            pltpu.make_async_copy(tab_hbm.at[c], tab_v, s_lo).wait()

        def start_idx(f, h, slot):
            pltpu.async_copy(idx_hbm.at[f, pl.ds(h * HB, HB)],
                             idx_bufs[slot], idx_sems[slot])

        def wait_idx(slot):
            pltpu.make_async_copy(idx_hbm.at[0, pl.ds(0, HB)],
                                  idx_bufs[slot], idx_sems[slot]).wait()

        def wait_out(slot):
            pltpu.make_async_copy(out_bufs[slot],
                                  out_hbm.at[c0, pl.ds(0, HB)], s_out).wait()

        def row_body(j, carry):
            c = c0 + j
            f = c // D
            f_nxt = (c + 1) // D
            wait_tab(c)
            for h in range(nh):
                slot = h % 2
                if h + 1 < nh:
                    start_idx(f, h + 1, (h + 1) % 2)
                else:
                    @pl.when(j < rows_per_w - 1)
                    def _():
                        start_idx(f_nxt, 0, (h + 1) % 2)
                wait_idx(slot)
                if h < 2:
                    @pl.when(j > 0)
                    def _():
                        wait_out(slot)
                else:
                    wait_out(slot)
                buf = idx_bufs[slot]
                obuf = out_bufs[slot]

                def body(kk, carry3, buf=buf, obuf=obuf):
                    base = kk * (UNROLL * LANES)
                    ivs = [buf[pl.ds(base + u * LANES, LANES)]
                           for u in range(UNROLL)]
                    gs = [plsc.load_gather(tab_v, [iv]) for iv in ivs]
                    for u in range(UNROLL):
                        obuf[pl.ds(base + u * LANES, LANES)] = gs[u]
                    return carry3

                lax.fori_loop(0, HB // (UNROLL * LANES), body, 0)
                if h == nh - 1:
                    @pl.when(j < rows_per_w - 1)
                    def _():
                        start_tab(c + 1)
                pltpu.async_copy(obuf, out_hbm.at[c, pl.ds(h * HB, HB)], s_out)
            return carry

        start_tab(c0)
        start_idx(c0 // D, 0, 0)
        lax.fori_loop(0, rows_per_w, row_body, 0)
        wait_out(0)
        wait_out(1)

    return k


def kernel(category_inputs, tables):
    B, F = category_inputs.shape
    _, V, D = tables.shape
    idx_t = category_inputs.astype(jnp.int32).T
    tab_t = jnp.transpose(tables, (0, 2, 1)).reshape(F * D, V)
    out_t = _make_gather_t(F * D, V, B, D)(tab_t, idx_t)
    return out_t.T
